# Initial kernel scaffold; baseline (speedup 1.0000x reference)
import functools

import jax
import jax.numpy as jnp
from jax import lax
from jax.experimental import pallas as pl
from jax.experimental.pallas import tpu as pltpu
from jax.experimental.pallas import tpu_sc as plsc

N_NODES = 10000
N_EDGES = 320000
D_FEAT = 128
NUM_LAYERS = 2

NC = 2
NS = 16
NW = NC * NS
LANES = 16

EDGE_CHUNK = 80
EDGE_ROWS = N_EDGES // EDGE_CHUNK
CHUNKS_PER_TILE = EDGE_ROWS // NW
NODES_PER_TILE = N_NODES // NS
ZERO_ROWS = 125


def _sc_layer_body(h_hbm, src_hbm, dst_hbm, ew_hbm, out_hbm,
                   src_v, dst_v, ew_v, rows_v, zero_v, accum_sh, sem):
    c = lax.axis_index("c")
    s = lax.axis_index("s")
    wid = c * NS + s

    z16 = jnp.zeros((LANES,), jnp.float32)

    @pl.loop(0, ZERO_ROWS)
    def _zero_fill(r):
        for j in range(D_FEAT // LANES):
            zero_v[r, pl.ds(j * LANES, LANES)] = z16

    for p in range(NODES_PER_TILE // ZERO_ROWS):
        pltpu.sync_copy(
            zero_v, accum_sh.at[pl.ds(s * NODES_PER_TILE + p * ZERO_ROWS,
                                      ZERO_ROWS)])
    plsc.subcore_barrier()

    base_row = wid * CHUNKS_PER_TILE

    @pl.loop(0, CHUNKS_PER_TILE)
    def _chunk(t):
        row = base_row + t
        pltpu.sync_copy(src_hbm.at[row], src_v)
        pltpu.sync_copy(dst_hbm.at[row], dst_v)
        pltpu.sync_copy(ew_hbm.at[row], ew_v)
        pltpu.async_copy(h_hbm.at[src_v], rows_v, sem).wait()

        @pl.loop(0, EDGE_CHUNK, unroll=8)
        def _scale(r):
            idx = r + jnp.zeros((LANES,), jnp.int32)
            w = plsc.load_gather(ew_v, [idx])
            for j in range(D_FEAT // LANES):
                sl = pl.ds(j * LANES, LANES)
                rows_v[r, sl] = rows_v[r, sl] * w

        pltpu.sync_copy(rows_v, accum_sh.at[dst_v], add=True)

    plsc.subcore_barrier()

    pltpu.sync_copy(accum_sh.at[pl.ds(s * NODES_PER_TILE, NODES_PER_TILE)],
                    out_hbm.at[c, pl.ds(s * NODES_PER_TILE, NODES_PER_TILE)])


@jax.jit
def _sc_layer(h, src2d, dst2d, ew2d):
    mesh = plsc.VectorSubcoreMesh(core_axis_name="c", subcore_axis_name="s")
    return pl.kernel(
        _sc_layer_body,
        out_type=jax.ShapeDtypeStruct((NC, N_NODES, D_FEAT), jnp.float32),
        mesh=mesh,
        scratch_types=[
            pltpu.VMEM((EDGE_CHUNK,), jnp.int32),
            pltpu.VMEM((EDGE_CHUNK,), jnp.int32),
            pltpu.VMEM((EDGE_CHUNK,), jnp.float32),
            pltpu.VMEM((EDGE_CHUNK, D_FEAT), jnp.float32),
            pltpu.VMEM((ZERO_ROWS, D_FEAT), jnp.float32),
            pltpu.VMEM_SHARED((N_NODES, D_FEAT), jnp.float32),
            pltpu.SemaphoreType.DMA,
        ],
    )(h, src2d, dst2d, ew2d)


def _combine_body(p_ref, o_ref):
    o_ref[...] = p_ref[0] + p_ref[1]


@jax.jit
def _combine(partials):
    return pl.pallas_call(
        _combine_body,
        out_shape=jax.ShapeDtypeStruct((N_NODES, D_FEAT), jnp.float32),
    )(partials)


def kernel(x, edge_index, edge_weight):
    src2d = edge_index[0].reshape(EDGE_ROWS, EDGE_CHUNK)
    dst2d = edge_index[1].reshape(EDGE_ROWS, EDGE_CHUNK)
    ew2d = edge_weight.reshape(EDGE_ROWS, EDGE_CHUNK)
    h = x
    for _ in range(NUM_LAYERS):
        partials = _sc_layer(h, src2d, dst2d, ew2d)
        h = _combine(partials)
    return h


# speedup vs baseline: 3.7701x; 3.7701x over previous
import functools

import jax
import jax.numpy as jnp
from jax import lax
from jax.experimental import pallas as pl
from jax.experimental.pallas import tpu as pltpu
from jax.experimental.pallas import tpu_sc as plsc

N_NODES = 10000
N_EDGES = 320000
D_FEAT = 128
NUM_LAYERS = 2

NC = 2
NS = 16
NW = NC * NS
LANES = 16

EDGE_CHUNK = 80
EDGE_ROWS = N_EDGES // EDGE_CHUNK
CHUNKS_PER_TILE = EDGE_ROWS // NW
PIECE = 80
N_PIECES = N_NODES // PIECE


def _sc_layer_body(h_hbm, src_hbm, dst_hbm, ew_hbm, out_hbm,
                   src_v, dst_v, ew_v, rows_v, zero_v, accum_sh, sem):
    c = lax.axis_index("c")
    s = lax.axis_index("s")
    wid = c * NS + s

    z16 = jnp.zeros((LANES,), jnp.float32)

    @pl.loop(0, PIECE)
    def _zero_fill(r):
        for j in range(D_FEAT // LANES):
            zero_v[r, pl.ds(j * LANES, LANES)] = z16

    for t in range((N_PIECES + NS - 1) // NS):
        p = t * NS + s

        @pl.when(p < N_PIECES)
        def _():
            off = pl.multiple_of(p * PIECE, 16)
            pltpu.sync_copy(zero_v, accum_sh.at[pl.ds(off, PIECE)])

    plsc.subcore_barrier()

    base_row = wid * CHUNKS_PER_TILE

    @pl.loop(0, CHUNKS_PER_TILE)
    def _chunk(t):
        row = base_row + t
        pltpu.sync_copy(src_hbm.at[row], src_v)
        pltpu.sync_copy(dst_hbm.at[row], dst_v)
        pltpu.sync_copy(ew_hbm.at[row], ew_v)
        pltpu.async_copy(h_hbm.at[src_v], rows_v, sem).wait()

        @pl.loop(0, EDGE_CHUNK, unroll=8)
        def _scale(r):
            idx = r + jnp.zeros((LANES,), jnp.int32)
            w = plsc.load_gather(ew_v, [idx])
            for j in range(D_FEAT // LANES):
                sl = pl.ds(j * LANES, LANES)
                rows_v[r, sl] = rows_v[r, sl] * w

        pltpu.sync_copy(rows_v, accum_sh.at[dst_v], add=True)

    plsc.subcore_barrier()

    for t in range((N_PIECES + NS - 1) // NS):
        p = t * NS + s

        @pl.when(p < N_PIECES)
        def _():
            off = pl.multiple_of(p * PIECE, 16)
            pltpu.sync_copy(accum_sh.at[pl.ds(off, PIECE)],
                            out_hbm.at[c, pl.ds(off, PIECE)])


@jax.jit
def _sc_layer(h, src2d, dst2d, ew2d):
    mesh = plsc.VectorSubcoreMesh(core_axis_name="c", subcore_axis_name="s")
    return pl.kernel(
        _sc_layer_body,
        out_type=jax.ShapeDtypeStruct((NC, N_NODES, D_FEAT), jnp.float32),
        mesh=mesh,
        compiler_params=pltpu.CompilerParams(needs_layout_passes=False),
        scratch_types=[
            pltpu.VMEM((EDGE_CHUNK,), jnp.int32),
            pltpu.VMEM((EDGE_CHUNK,), jnp.int32),
            pltpu.VMEM((EDGE_CHUNK,), jnp.float32),
            pltpu.VMEM((EDGE_CHUNK, D_FEAT), jnp.float32),
            pltpu.VMEM((PIECE, D_FEAT), jnp.float32),
            pltpu.VMEM_SHARED((N_NODES, D_FEAT), jnp.float32),
            pltpu.SemaphoreType.DMA,
        ],
    )(h, src2d, dst2d, ew2d)


def _combine_body(p_ref, o_ref):
    o_ref[...] = p_ref[0] + p_ref[1]


@jax.jit
def _combine(partials):
    return pl.pallas_call(
        _combine_body,
        out_shape=jax.ShapeDtypeStruct((N_NODES, D_FEAT), jnp.float32),
    )(partials)


def kernel(x, edge_index, edge_weight):
    src2d = edge_index[0].reshape(EDGE_ROWS, EDGE_CHUNK)
    dst2d = edge_index[1].reshape(EDGE_ROWS, EDGE_CHUNK)
    ew2d = edge_weight.reshape(EDGE_ROWS, EDGE_CHUNK)
    h = x
    for _ in range(NUM_LAYERS):
        partials = _sc_layer(h, src2d, dst2d, ew2d)
        h = _combine(partials)
    return h


# speedup vs baseline: 9.2994x; 2.4666x over previous
import functools

import jax
import jax.numpy as jnp
from jax import lax
from jax.experimental import pallas as pl
from jax.experimental.pallas import tpu as pltpu
from jax.experimental.pallas import tpu_sc as plsc

N_NODES = 10000
N_EDGES = 320000
D_FEAT = 128
NUM_LAYERS = 2

NC = 2
NS = 16
NW = NC * NS
LANES = 16

EDGE_CHUNK = 80
EDGE_ROWS = N_EDGES // EDGE_CHUNK
CHUNKS_PER_TILE = EDGE_ROWS // NW
PIECE = 80
N_PIECES = N_NODES // PIECE


def _sc_layer_body(h_hbm, pack_hbm, out_hbm,
                   pk0_v, pk1_v, pk2_v, pk3_v, rows0_v, rows1_v, accum_sh,
                   isem0, isem1, isem2, isem3, gsem0, gsem1):
    c = lax.axis_index("c")
    s = lax.axis_index("s")
    wid = c * NS + s

    z16 = jnp.zeros((LANES,), jnp.float32)

    @pl.loop(0, PIECE)
    def _zero_fill(r):
        for j in range(D_FEAT // LANES):
            rows0_v[r, pl.ds(j * LANES, LANES)] = z16

    for t in range((N_PIECES + NS - 1) // NS):
        p = t * NS + s

        @pl.when(p < N_PIECES)
        def _():
            off = pl.multiple_of(p * PIECE, 16)
            pltpu.sync_copy(rows0_v, accum_sh.at[pl.ds(off, PIECE)])

    plsc.subcore_barrier()

    pks = (pk0_v, pk1_v, pk2_v, pk3_v)
    rows = (rows0_v, rows1_v)
    isems = (isem0, isem1, isem2, isem3)
    gsems = (gsem0, gsem1)

    def idx_start(t, p):
        pltpu.async_copy(pack_hbm.at[wid, t], pks[p], isems[p])

    def idx_wait(p):
        pltpu.make_async_copy(pack_hbm.at[wid, 0], pks[p], isems[p]).wait()

    def gather_start(p, b):
        pltpu.async_copy(h_hbm.at[pks[p].at[0]], rows[b], gsems[b])

    def gather_wait(p, b):
        pltpu.make_async_copy(h_hbm.at[pks[p].at[0]], rows[b], gsems[b]).wait()

    def scale_and_scatter(p, b):
        buf = rows[b]

        @pl.loop(0, EDGE_CHUNK, unroll=8)
        def _scale(r):
            idx = r + jnp.zeros((LANES,), jnp.int32)
            wbits = plsc.load_gather(pks[p].at[2], [idx])
            w = plsc.bitcast(wbits, jnp.float32)
            for j in range(D_FEAT // LANES):
                sl = pl.ds(j * LANES, LANES)
                buf[r, sl] = buf[r, sl] * w

        pltpu.sync_copy(buf, accum_sh.at[pks[p].at[1]], add=True)

    for p in range(4):
        idx_start(p, p)
    idx_wait(0)
    gather_start(0, 0)
    idx_wait(1)
    gather_start(1, 1)

    @pl.loop(0, CHUNKS_PER_TILE, step=4)
    def _round(t0):
        for u in range(4):
            t = t0 + u
            p = u
            b = u % 2

            @pl.when(t < CHUNKS_PER_TILE)
            def _():
                gather_wait(p, b)
                scale_and_scatter(p, b)

                @pl.when(t + 4 < CHUNKS_PER_TILE)
                def _():
                    idx_start(t + 4, p)

                @pl.when(t + 2 < CHUNKS_PER_TILE)
                def _():
                    idx_wait((p + 2) % 4)
                    gather_start((p + 2) % 4, b)

    plsc.subcore_barrier()

    for t in range((N_PIECES + NS - 1) // NS):
        p = t * NS + s

        @pl.when(p < N_PIECES)
        def _():
            off = pl.multiple_of(p * PIECE, 16)
            pltpu.sync_copy(accum_sh.at[pl.ds(off, PIECE)],
                            out_hbm.at[c, pl.ds(off, PIECE)])


@jax.jit
def _sc_layer(h, pack):
    mesh = plsc.VectorSubcoreMesh(core_axis_name="c", subcore_axis_name="s")
    return pl.kernel(
        _sc_layer_body,
        out_type=jax.ShapeDtypeStruct((NC, N_NODES, D_FEAT), jnp.float32),
        mesh=mesh,
        compiler_params=pltpu.CompilerParams(needs_layout_passes=False),
        scratch_types=[
            pltpu.VMEM((3, EDGE_CHUNK), jnp.int32),
            pltpu.VMEM((3, EDGE_CHUNK), jnp.int32),
            pltpu.VMEM((3, EDGE_CHUNK), jnp.int32),
            pltpu.VMEM((3, EDGE_CHUNK), jnp.int32),
            pltpu.VMEM((EDGE_CHUNK, D_FEAT), jnp.float32),
            pltpu.VMEM((EDGE_CHUNK, D_FEAT), jnp.float32),
            pltpu.VMEM_SHARED((N_NODES, D_FEAT), jnp.float32),
            pltpu.SemaphoreType.DMA,
            pltpu.SemaphoreType.DMA,
            pltpu.SemaphoreType.DMA,
            pltpu.SemaphoreType.DMA,
            pltpu.SemaphoreType.DMA,
            pltpu.SemaphoreType.DMA,
        ],
    )(h, pack)


def _combine_body(p_ref, o_ref):
    o_ref[...] = p_ref[0] + p_ref[1]


@jax.jit
def _combine(partials):
    return pl.pallas_call(
        _combine_body,
        out_shape=jax.ShapeDtypeStruct((N_NODES, D_FEAT), jnp.float32),
    )(partials)


def kernel(x, edge_index, edge_weight):
    src3d = edge_index[0].reshape(NW, CHUNKS_PER_TILE, EDGE_CHUNK)
    dst3d = edge_index[1].reshape(NW, CHUNKS_PER_TILE, EDGE_CHUNK)
    ewbits = jax.lax.bitcast_convert_type(
        edge_weight, jnp.int32).reshape(NW, CHUNKS_PER_TILE, EDGE_CHUNK)
    pack = jnp.stack([src3d, dst3d, ewbits], axis=2)
    h = x
    for _ in range(NUM_LAYERS):
        partials = _sc_layer(h, pack)
        h = _combine(partials)
    return h


# speedup vs baseline: 10.3850x; 1.1167x over previous
import functools

import jax
import jax.numpy as jnp
from jax import lax
from jax.experimental import pallas as pl
from jax.experimental.pallas import tpu as pltpu
from jax.experimental.pallas import tpu_sc as plsc

N_NODES = 10000
N_EDGES = 320000
D_FEAT = 128
NUM_LAYERS = 2

NC = 2
NS = 16
NW = NC * NS
LANES = 16

EDGE_CHUNK = 80
EDGE_ROWS = N_EDGES // EDGE_CHUNK
CHUNKS_PER_TILE = EDGE_ROWS // NW
PIECE = 80
N_PIECES = N_NODES // PIECE


NPK = 6
NRW = 4
LCM = 12


def _sc_layer_body(h_hbm, pack_hbm, out_hbm,
                   pks, rows, accum_sh, isems, gsems, ssems):
    c = lax.axis_index("c")
    s = lax.axis_index("s")
    wid = c * NS + s

    z16 = jnp.zeros((LANES,), jnp.float32)

    @pl.loop(0, PIECE)
    def _zero_fill(r):
        for j in range(D_FEAT // LANES):
            rows[0][r, pl.ds(j * LANES, LANES)] = z16

    for t in range((N_PIECES + NS - 1) // NS):
        p = t * NS + s

        @pl.when(p < N_PIECES)
        def _():
            off = pl.multiple_of(p * PIECE, 16)
            pltpu.sync_copy(rows[0], accum_sh.at[pl.ds(off, PIECE)])

    plsc.subcore_barrier()

    CPT = CHUNKS_PER_TILE

    def idx_start(t, p):
        pltpu.async_copy(pack_hbm.at[wid, t], pks[p], isems[p])

    def idx_wait(p):
        pltpu.make_async_copy(pack_hbm.at[wid, 0], pks[p], isems[p]).wait()

    def gather_start(p, m):
        pltpu.async_copy(h_hbm.at[pks[p].at[0]], rows[m], gsems[m])

    def gather_wait(p, m):
        pltpu.make_async_copy(h_hbm.at[pks[p].at[0]], rows[m], gsems[m]).wait()

    def scatter_start(p, m):
        pltpu.async_copy(rows[m], accum_sh.at[pks[p].at[1]], ssems[m],
                         add=True)

    def scatter_wait(p, m):
        pltpu.make_async_copy(rows[m], accum_sh.at[pks[p].at[1]],
                              ssems[m]).wait()

    def scale(p, m):
        buf = rows[m]

        @pl.loop(0, EDGE_CHUNK, unroll=8)
        def _scale(r):
            idx = r + jnp.zeros((LANES,), jnp.int32)
            wbits = plsc.load_gather(pks[p].at[2], [idx])
            w = plsc.bitcast(wbits, jnp.float32)
            for j in range(D_FEAT // LANES):
                sl = pl.ds(j * LANES, LANES)
                buf[r, sl] = buf[r, sl] * w

    for p in range(NPK):
        idx_start(p, p)
    for t in range(2):
        idx_wait(t)
        gather_start(t, t)

    @pl.loop(0, (CPT + LCM - 1) // LCM * LCM, step=LCM)
    def _round(t0):
        for u in range(LCM):
            t = t0 + u
            p = u % NPK
            m = u % NRW

            @pl.when(t < CPT)
            def _():
                gather_wait(p, m)
                scale(p, m)
                scatter_start(p, m)

                @pl.when(t >= 2)
                def _():
                    scatter_wait((p - 2) % NPK, (m - 2) % NRW)

                @pl.when(t + 4 < CPT)
                def _():
                    idx_start(t + 4, (p + 4) % NPK)

                @pl.when(t + 2 < CPT)
                def _():
                    idx_wait((p + 2) % NPK)
                    gather_start((p + 2) % NPK, (m + 2) % NRW)

    scatter_wait((CPT - 2) % NPK, (CPT - 2) % NRW)
    scatter_wait((CPT - 1) % NPK, (CPT - 1) % NRW)

    plsc.subcore_barrier()

    for t in range((N_PIECES + NS - 1) // NS):
        p = t * NS + s

        @pl.when(p < N_PIECES)
        def _():
            off = pl.multiple_of(p * PIECE, 16)
            pltpu.sync_copy(accum_sh.at[pl.ds(off, PIECE)],
                            out_hbm.at[c, pl.ds(off, PIECE)])


@jax.jit
def _sc_layer(h, pack):
    mesh = plsc.VectorSubcoreMesh(core_axis_name="c", subcore_axis_name="s")
    return pl.kernel(
        _sc_layer_body,
        out_type=jax.ShapeDtypeStruct((NC, N_NODES, D_FEAT), jnp.float32),
        mesh=mesh,
        compiler_params=pltpu.CompilerParams(needs_layout_passes=False),
        scratch_types=[
            [pltpu.VMEM((3, EDGE_CHUNK), jnp.int32) for _ in range(NPK)],
            [pltpu.VMEM((EDGE_CHUNK, D_FEAT), jnp.float32)
             for _ in range(NRW)],
            pltpu.VMEM_SHARED((N_NODES, D_FEAT), jnp.float32),
            [pltpu.SemaphoreType.DMA for _ in range(NPK)],
            [pltpu.SemaphoreType.DMA for _ in range(NRW)],
            [pltpu.SemaphoreType.DMA for _ in range(NRW)],
        ],
    )(h, pack)


def _combine_body(p_ref, o_ref):
    o_ref[...] = p_ref[0] + p_ref[1]


@jax.jit
def _combine(partials):
    return pl.pallas_call(
        _combine_body,
        out_shape=jax.ShapeDtypeStruct((N_NODES, D_FEAT), jnp.float32),
    )(partials)


def kernel(x, edge_index, edge_weight):
    src3d = edge_index[0].reshape(NW, CHUNKS_PER_TILE, EDGE_CHUNK)
    dst3d = edge_index[1].reshape(NW, CHUNKS_PER_TILE, EDGE_CHUNK)
    ewbits = jax.lax.bitcast_convert_type(
        edge_weight, jnp.int32).reshape(NW, CHUNKS_PER_TILE, EDGE_CHUNK)
    pack = jnp.stack([src3d, dst3d, ewbits], axis=2)
    h = x
    for _ in range(NUM_LAYERS):
        partials = _sc_layer(h, pack)
        h = _combine(partials)
    return h


# speedup vs baseline: 11.3628x; 1.0942x over previous
import functools

import jax
import jax.numpy as jnp
from jax import lax
from jax.experimental import pallas as pl
from jax.experimental.pallas import tpu as pltpu
from jax.experimental.pallas import tpu_sc as plsc

N_NODES = 10000
N_EDGES = 320000
D_FEAT = 128
NUM_LAYERS = 2

NC = 2
NS = 16
NW = NC * NS
LANES = 16

EDGE_CHUNK = 80
EDGE_ROWS = N_EDGES // EDGE_CHUNK
CHUNKS_PER_TILE = EDGE_ROWS // NW
PIECE = 80
N_PIECES = N_NODES // PIECE


NPK = 6
NRW = 4
LCM = 12


def _sc_layer_body(h_hbm, pack_hbm, out_hbm,
                   pks, rows, accum_sh, isems, gsems, ssems):
    c = lax.axis_index("c")
    s = lax.axis_index("s")
    wid = c * NS + s

    z16 = jnp.zeros((LANES,), jnp.float32)

    @pl.loop(0, PIECE)
    def _zero_fill(r):
        for j in range(D_FEAT // LANES):
            rows[0][r, pl.ds(j * LANES, LANES)] = z16

    for t in range((N_PIECES + NS - 1) // NS):
        p = t * NS + s

        @pl.when(p < N_PIECES)
        def _():
            off = pl.multiple_of(p * PIECE, 16)
            pltpu.sync_copy(rows[0], accum_sh.at[pl.ds(off, PIECE)])

    plsc.subcore_barrier()

    CPT = CHUNKS_PER_TILE

    def idx_start(t, p):
        pltpu.async_copy(pack_hbm.at[wid, t], pks[p], isems[p])

    def idx_wait(p):
        pltpu.make_async_copy(pack_hbm.at[wid, 0], pks[p], isems[p]).wait()

    def gather_start(p, m):
        pltpu.async_copy(h_hbm.at[pks[p].at[0]], rows[m], gsems[m])

    def gather_wait(p, m):
        pltpu.make_async_copy(h_hbm.at[pks[p].at[0]], rows[m], gsems[m]).wait()

    def scatter_start(p, m):
        pltpu.async_copy(rows[m], accum_sh.at[pks[p].at[1]], ssems[m],
                         add=True)

    def scatter_wait(p, m):
        pltpu.make_async_copy(rows[m], accum_sh.at[pks[p].at[1]],
                              ssems[m]).wait()

    def scale(p, m):
        buf = rows[m]

        @plsc.parallel_loop(0, EDGE_CHUNK, unroll=8)
        def _scale(r):
            idx = r + jnp.zeros((LANES,), jnp.int32)
            wbits = plsc.load_gather(pks[p].at[2], [idx])
            w = plsc.bitcast(wbits, jnp.float32)
            for j in range(D_FEAT // LANES):
                sl = pl.ds(j * LANES, LANES)
                buf[r, sl] = buf[r, sl] * w

    for p in range(NPK):
        idx_start(p, p)
    for t in range(2):
        idx_wait(t)
        gather_start(t, t)

    @pl.loop(0, (CPT + LCM - 1) // LCM * LCM, step=LCM)
    def _round(t0):
        for u in range(LCM):
            t = t0 + u
            p = u % NPK
            m = u % NRW

            @pl.when(t < CPT)
            def _():
                gather_wait(p, m)
                scale(p, m)
                scatter_start(p, m)

                @pl.when(t >= 2)
                def _():
                    scatter_wait((p - 2) % NPK, (m - 2) % NRW)

                @pl.when(t + 4 < CPT)
                def _():
                    idx_start(t + 4, (p + 4) % NPK)

                @pl.when(t + 2 < CPT)
                def _():
                    idx_wait((p + 2) % NPK)
                    gather_start((p + 2) % NPK, (m + 2) % NRW)

    scatter_wait((CPT - 2) % NPK, (CPT - 2) % NRW)
    scatter_wait((CPT - 1) % NPK, (CPT - 1) % NRW)

    plsc.subcore_barrier()

    for t in range((N_PIECES + NS - 1) // NS):
        p = t * NS + s

        @pl.when(p < N_PIECES)
        def _():
            off = pl.multiple_of(p * PIECE, 16)
            pltpu.sync_copy(accum_sh.at[pl.ds(off, PIECE)],
                            out_hbm.at[c, pl.ds(off, PIECE)])


@jax.jit
def _sc_layer(h, pack):
    mesh = plsc.VectorSubcoreMesh(core_axis_name="c", subcore_axis_name="s")
    return pl.kernel(
        _sc_layer_body,
        out_type=jax.ShapeDtypeStruct((NC, N_NODES, D_FEAT), jnp.float32),
        mesh=mesh,
        compiler_params=pltpu.CompilerParams(needs_layout_passes=False),
        scratch_types=[
            [pltpu.VMEM((3, EDGE_CHUNK), jnp.int32) for _ in range(NPK)],
            [pltpu.VMEM((EDGE_CHUNK, D_FEAT), jnp.float32)
             for _ in range(NRW)],
            pltpu.VMEM_SHARED((N_NODES, D_FEAT), jnp.float32),
            [pltpu.SemaphoreType.DMA for _ in range(NPK)],
            [pltpu.SemaphoreType.DMA for _ in range(NRW)],
            [pltpu.SemaphoreType.DMA for _ in range(NRW)],
        ],
    )(h, pack)


def _combine_body(p_ref, o_ref):
    o_ref[...] = p_ref[0] + p_ref[1]


@jax.jit
def _combine(partials):
    return pl.pallas_call(
        _combine_body,
        out_shape=jax.ShapeDtypeStruct((N_NODES, D_FEAT), jnp.float32),
    )(partials)


def kernel(x, edge_index, edge_weight):
    src3d = edge_index[0].reshape(NW, CHUNKS_PER_TILE, EDGE_CHUNK)
    dst3d = edge_index[1].reshape(NW, CHUNKS_PER_TILE, EDGE_CHUNK)
    ewbits = jax.lax.bitcast_convert_type(
        edge_weight, jnp.int32).reshape(NW, CHUNKS_PER_TILE, EDGE_CHUNK)
    pack = jnp.stack([src3d, dst3d, ewbits], axis=2)
    h = x
    for _ in range(NUM_LAYERS):
        partials = _sc_layer(h, pack)
        h = _combine(partials)
    return h


# speedup vs baseline: 11.8540x; 1.0432x over previous
import functools

import jax
import jax.numpy as jnp
from jax import lax
from jax.experimental import pallas as pl
from jax.experimental.pallas import tpu as pltpu
from jax.experimental.pallas import tpu_sc as plsc

N_NODES = 10000
N_EDGES = 320000
D_FEAT = 128
NUM_LAYERS = 2

NC = 2
NS = 16
NW = NC * NS
LANES = 16

EDGE_CHUNK = 80
EDGE_ROWS = N_EDGES // EDGE_CHUNK
CHUNKS_PER_TILE = EDGE_ROWS // NW
PIECE = 80
N_PIECES = N_NODES // PIECE


NPK = 6
NRW = 4
LCM = 12


def _sc_layer_body(h_hbm, pack_hbm, out_hbm,
                   pks, rows, accum_sh, isems, gsems, ssems):
    c = lax.axis_index("c")
    s = lax.axis_index("s")
    wid = c * NS + s

    z16 = jnp.zeros((LANES,), jnp.float32)

    @pl.loop(0, PIECE)
    def _zero_fill(r):
        for j in range(D_FEAT // LANES):
            rows[0][r, pl.ds(j * LANES, LANES)] = z16

    for t in range((N_PIECES + NS - 1) // NS):
        p = t * NS + s

        @pl.when(p < N_PIECES)
        def _():
            off = pl.multiple_of(p * PIECE, 16)
            pltpu.sync_copy(rows[0], accum_sh.at[pl.ds(off, PIECE)])

    plsc.subcore_barrier()

    CPT = CHUNKS_PER_TILE

    def idx_start(t, p):
        pltpu.async_copy(pack_hbm.at[wid, t], pks[p], isems[p])

    def idx_wait(p):
        pltpu.make_async_copy(pack_hbm.at[wid, 0], pks[p], isems[p]).wait()

    def gather_start(p, m):
        pltpu.async_copy(h_hbm.at[pks[p].at[0]], rows[m], gsems[m])

    def gather_wait(p, m):
        pltpu.make_async_copy(h_hbm.at[pks[p].at[0]], rows[m], gsems[m]).wait()

    def scatter_start(p, m):
        pltpu.async_copy(rows[m], accum_sh.at[pks[p].at[1]], ssems[m],
                         add=True)

    def scatter_wait(p, m):
        pltpu.make_async_copy(rows[m], accum_sh.at[pks[p].at[1]],
                              ssems[m]).wait()

    def scale(p, m):
        buf = rows[m]

        @plsc.parallel_loop(0, EDGE_CHUNK, unroll=8)
        def _scale(r):
            idx = r + jnp.zeros((LANES,), jnp.int32)
            wbits = plsc.load_gather(pks[p].at[2], [idx])
            w = plsc.bitcast(wbits, jnp.float32)
            for j in range(D_FEAT // LANES):
                sl = pl.ds(j * LANES, LANES)
                buf[r, sl] = buf[r, sl] * w

    for p in range(NPK):
        idx_start(p, p)
    for t in range(2):
        idx_wait(t)
        gather_start(t, t)

    @pl.loop(0, (CPT + LCM - 1) // LCM * LCM, step=LCM)
    def _round(t0):
        for u in range(LCM):
            t = t0 + u
            p = u % NPK
            m = u % NRW

            @pl.when(t < CPT)
            def _():
                gather_wait(p, m)

                @pl.when(t >= 2)
                def _():
                    scatter_wait((p - 2) % NPK, (m - 2) % NRW)

                @pl.when(t + 2 < CPT)
                def _():
                    idx_wait((p + 2) % NPK)
                    gather_start((p + 2) % NPK, (m + 2) % NRW)

                @pl.when(t + 4 < CPT)
                def _():
                    idx_start(t + 4, (p + 4) % NPK)

                scale(p, m)
                scatter_start(p, m)

    scatter_wait((CPT - 2) % NPK, (CPT - 2) % NRW)
    scatter_wait((CPT - 1) % NPK, (CPT - 1) % NRW)

    plsc.subcore_barrier()

    for t in range((N_PIECES + NS - 1) // NS):
        p = t * NS + s

        @pl.when(p < N_PIECES)
        def _():
            off = pl.multiple_of(p * PIECE, 16)
            pltpu.sync_copy(accum_sh.at[pl.ds(off, PIECE)],
                            out_hbm.at[c, pl.ds(off, PIECE)])


@jax.jit
def _sc_layer(h, pack):
    mesh = plsc.VectorSubcoreMesh(core_axis_name="c", subcore_axis_name="s")
    return pl.kernel(
        _sc_layer_body,
        out_type=jax.ShapeDtypeStruct((NC, N_NODES, D_FEAT), jnp.float32),
        mesh=mesh,
        compiler_params=pltpu.CompilerParams(needs_layout_passes=False),
        scratch_types=[
            [pltpu.VMEM((3, EDGE_CHUNK), jnp.int32) for _ in range(NPK)],
            [pltpu.VMEM((EDGE_CHUNK, D_FEAT), jnp.float32)
             for _ in range(NRW)],
            pltpu.VMEM_SHARED((N_NODES, D_FEAT), jnp.float32),
            [pltpu.SemaphoreType.DMA for _ in range(NPK)],
            [pltpu.SemaphoreType.DMA for _ in range(NRW)],
            [pltpu.SemaphoreType.DMA for _ in range(NRW)],
        ],
    )(h, pack)


def _combine_body(p_ref, o_ref):
    o_ref[...] = p_ref[0] + p_ref[1]


@jax.jit
def _combine(partials):
    return pl.pallas_call(
        _combine_body,
        out_shape=jax.ShapeDtypeStruct((N_NODES, D_FEAT), jnp.float32),
    )(partials)


def kernel(x, edge_index, edge_weight):
    src3d = edge_index[0].reshape(NW, CHUNKS_PER_TILE, EDGE_CHUNK)
    dst3d = edge_index[1].reshape(NW, CHUNKS_PER_TILE, EDGE_CHUNK)
    ewbits = jax.lax.bitcast_convert_type(
        edge_weight, jnp.int32).reshape(NW, CHUNKS_PER_TILE, EDGE_CHUNK)
    pack = jnp.stack([src3d, dst3d, ewbits], axis=2)
    h = x
    for _ in range(NUM_LAYERS):
        partials = _sc_layer(h, pack)
        h = _combine(partials)
    return h


# speedup vs baseline: 11.9677x; 1.0096x over previous
import functools

import jax
import jax.numpy as jnp
from jax import lax
from jax.experimental import pallas as pl
from jax.experimental.pallas import tpu as pltpu
from jax.experimental.pallas import tpu_sc as plsc

N_NODES = 10000
N_EDGES = 320000
D_FEAT = 128
NUM_LAYERS = 2

NC = 2
NS = 16
NW = NC * NS
LANES = 16

EDGE_CHUNK = 80
EDGE_ROWS = N_EDGES // EDGE_CHUNK
CHUNKS_PER_TILE = EDGE_ROWS // NW
PIECE = 80
N_PIECES = N_NODES // PIECE


NPK = 6
NRW = 4
LCM = 12


def _sc_layer_body(h_hbm, pack_hbm, out_hbm,
                   pks, rows, accum_sh, isems, gsems, ssems):
    c = lax.axis_index("c")
    s = lax.axis_index("s")
    wid = c * NS + s

    CPT = CHUNKS_PER_TILE

    def idx_start(t, p):
        pltpu.async_copy(pack_hbm.at[wid, t], pks[p], isems[p])

    def idx_wait(p):
        pltpu.make_async_copy(pack_hbm.at[wid, 0], pks[p], isems[p]).wait()

    def gather_start(p, m):
        pltpu.async_copy(h_hbm.at[pks[p].at[0]], rows[m], gsems[m])

    def gather_wait(p, m):
        pltpu.make_async_copy(h_hbm.at[pks[p].at[0]], rows[m], gsems[m]).wait()

    def scatter_start(p, m):
        pltpu.async_copy(rows[m], accum_sh.at[pks[p].at[1]], ssems[m],
                         add=True)

    def scatter_wait(p, m):
        pltpu.make_async_copy(rows[m], accum_sh.at[pks[p].at[1]],
                              ssems[m]).wait()

    def scale(p, m):
        buf = rows[m]

        @plsc.parallel_loop(0, EDGE_CHUNK, unroll=8)
        def _scale(r):
            idx = r + jnp.zeros((LANES,), jnp.int32)
            wbits = plsc.load_gather(pks[p].at[2], [idx])
            w = plsc.bitcast(wbits, jnp.float32)
            for j in range(D_FEAT // LANES):
                sl = pl.ds(j * LANES, LANES)
                buf[r, sl] = buf[r, sl] * w

    for p in range(NPK):
        idx_start(p, p)
    for t in range(2):
        idx_wait(t)
        gather_start(t, t + 2)

    z16 = jnp.zeros((LANES,), jnp.float32)

    @pl.loop(0, PIECE)
    def _zero_fill(r):
        for j in range(D_FEAT // LANES):
            rows[0][r, pl.ds(j * LANES, LANES)] = z16

    for tz in range((N_PIECES + NS - 1) // NS):
        pz = tz * NS + s

        @pl.when(pz < N_PIECES)
        def _():
            off = pl.multiple_of(pz * PIECE, 16)
            pltpu.sync_copy(rows[0], accum_sh.at[pl.ds(off, PIECE)])

    plsc.subcore_barrier()

    @pl.loop(0, (CPT + LCM - 1) // LCM * LCM, step=LCM)
    def _round(t0):
        for u in range(LCM):
            t = t0 + u
            p = u % NPK
            m = (u + 2) % NRW

            @pl.when(t < CPT)
            def _():
                gather_wait(p, m)

                @pl.when(t >= 2)
                def _():
                    scatter_wait((p - 2) % NPK, (m - 2) % NRW)

                @pl.when(t + 2 < CPT)
                def _():
                    idx_wait((p + 2) % NPK)
                    gather_start((p + 2) % NPK, (m + 2) % NRW)

                @pl.when(t + 4 < CPT)
                def _():
                    idx_start(t + 4, (p + 4) % NPK)

                scale(p, m)
                scatter_start(p, m)

    scatter_wait((CPT - 2) % NPK, (CPT - 2 + 2) % NRW)
    scatter_wait((CPT - 1) % NPK, (CPT - 1 + 2) % NRW)

    plsc.subcore_barrier()

    for t in range((N_PIECES + NS - 1) // NS):
        p = t * NS + s

        @pl.when(p < N_PIECES)
        def _():
            off = pl.multiple_of(p * PIECE, 16)
            pltpu.sync_copy(accum_sh.at[pl.ds(off, PIECE)],
                            out_hbm.at[c, pl.ds(off, PIECE)])


@jax.jit
def _sc_layer(h, pack):
    mesh = plsc.VectorSubcoreMesh(core_axis_name="c", subcore_axis_name="s")
    return pl.kernel(
        _sc_layer_body,
        out_type=jax.ShapeDtypeStruct((NC, N_NODES, D_FEAT), jnp.float32),
        mesh=mesh,
        compiler_params=pltpu.CompilerParams(needs_layout_passes=False),
        scratch_types=[
            [pltpu.VMEM((3, EDGE_CHUNK), jnp.int32) for _ in range(NPK)],
            [pltpu.VMEM((EDGE_CHUNK, D_FEAT), jnp.float32)
             for _ in range(NRW)],
            pltpu.VMEM_SHARED((N_NODES, D_FEAT), jnp.float32),
            [pltpu.SemaphoreType.DMA for _ in range(NPK)],
            [pltpu.SemaphoreType.DMA for _ in range(NRW)],
            [pltpu.SemaphoreType.DMA for _ in range(NRW)],
        ],
    )(h, pack)


def _combine_body(p_ref, o_ref):
    o_ref[...] = p_ref[0] + p_ref[1]


@jax.jit
def _combine(partials):
    return pl.pallas_call(
        _combine_body,
        out_shape=jax.ShapeDtypeStruct((N_NODES, D_FEAT), jnp.float32),
    )(partials)


def kernel(x, edge_index, edge_weight):
    src3d = edge_index[0].reshape(NW, CHUNKS_PER_TILE, EDGE_CHUNK)
    dst3d = edge_index[1].reshape(NW, CHUNKS_PER_TILE, EDGE_CHUNK)
    ewbits = jax.lax.bitcast_convert_type(
        edge_weight, jnp.int32).reshape(NW, CHUNKS_PER_TILE, EDGE_CHUNK)
    pack = jnp.stack([src3d, dst3d, ewbits], axis=2)
    h = x
    for _ in range(NUM_LAYERS):
        partials = _sc_layer(h, pack)
        h = _combine(partials)
    return h
